# Initial kernel scaffold; baseline (speedup 1.0000x reference)
import jax
import jax.numpy as jnp
from jax.experimental import pallas as pl


def kernel(x, edge_index, W1_l, W1_r, att1, b1, W2_l, W2_r, att2, b2):
    raise NotImplementedError("write your pallas kernel here")


# speedup vs baseline: 1.1783x; 1.1783x over previous
import functools

import jax
import jax.numpy as jnp
from jax.experimental import pallas as pl


def _mm2_kernel(x_ref, wl_ref, wr_ref, ol_ref, or_ref):
    x = x_ref[...]
    ol_ref[...] = jnp.dot(x, wl_ref[...], preferred_element_type=jnp.float32)
    or_ref[...] = jnp.dot(x, wr_ref[...], preferred_element_type=jnp.float32)


def _mm2(x, wl, wr):
    n = x.shape[0]
    f = wl.shape[1]
    return pl.pallas_call(
        _mm2_kernel,
        out_shape=[jax.ShapeDtypeStruct((n, f), jnp.float32)] * 2,
    )(x, wl, wr)


def _layer(x, src, dst, Wl, Wr, att, b, H, C, concat):
    Nn = x.shape[0]
    xl, xr = _mm2(x, Wl, Wr)
    xl = xl.reshape(Nn, H, C)
    xr = xr.reshape(Nn, H, C)
    m = xl[src] + xr[dst]
    e = jax.nn.leaky_relu(m, 0.2)
    logits = jnp.einsum('ehc,hc->eh', e, att)
    ex = jnp.exp(logits)
    denom = jax.ops.segment_sum(ex, dst, num_segments=Nn)
    msg = xl[src] * ex[:, :, None]
    out = jax.ops.segment_sum(msg, dst, num_segments=Nn)
    out = out / (denom[:, :, None] + 1e-16)
    if concat:
        out = out.reshape(Nn, H * C)
    else:
        out = out.mean(axis=1)
    return out + b


def kernel(x, edge_index, W1_l, W1_r, att1, b1, W2_l, W2_r, att2, b2):
    N = x.shape[0]
    loop = jnp.arange(N, dtype=edge_index.dtype)
    src = jnp.concatenate([edge_index[0], loop])
    dst = jnp.concatenate([edge_index[1], loop])
    h = _layer(x, src, dst, W1_l, W1_r, att1, b1, 8, 8, True)
    h = jax.nn.elu(h)
    h = _layer(h, src, dst, W2_l, W2_r, att2, b2, 1, 64, False)
    h = jax.nn.elu(h)
    return jax.nn.log_softmax(h, axis=-1)


# speedup vs baseline: 13.9979x; 11.8798x over previous
import functools

import jax
import jax.numpy as jnp
from jax import lax
from jax.experimental import pallas as pl
from jax.experimental.pallas import tpu as pltpu
from jax.experimental.pallas import tpu_sc as plsc

_F = 64
_CH = 128
_NW = 32


def _mm2_kernel(x_ref, wl_ref, wr_ref, ol_ref, or_ref):
    x = x_ref[...]
    ol_ref[...] = jnp.dot(x, wl_ref[...], preferred_element_type=jnp.float32)
    or_ref[...] = jnp.dot(x, wr_ref[...], preferred_element_type=jnp.float32)


def _mm2(x, wl, wr):
    n = x.shape[0]
    f = wl.shape[1]
    return pl.pallas_call(
        _mm2_kernel,
        out_shape=[jax.ShapeDtypeStruct((n, f), jnp.float32)] * 2,
    )(x, wl, wr)


def _sc_edge_pass(H, NP, n_chunks, xl, xr, src, dst, att, zmsg, zden):
    CH = _CH
    EPT = n_chunks * CH
    RPT = NP // 16
    CPH = _F // H
    mesh = plsc.VectorSubcoreMesh(core_axis_name="c", subcore_axis_name="s")

    @functools.partial(
        pl.kernel,
        mesh=mesh,
        out_type=[
            jax.ShapeDtypeStruct((2, NP, _F), jnp.float32),
            jax.ShapeDtypeStruct((2, NP, 16), jnp.float32),
        ],
        scratch_types=[
            pltpu.VMEM((CH,), jnp.int32),
            pltpu.VMEM((CH,), jnp.int32),
            pltpu.VMEM((CH, _F), jnp.float32),
            pltpu.VMEM((CH, _F), jnp.float32),
            pltpu.VMEM((CH, _F), jnp.float32),
            pltpu.VMEM((CH, 16), jnp.float32),
            pltpu.VMEM((_F,), jnp.float32),
            pltpu.VMEM_SHARED((NP, _F), jnp.float32),
            pltpu.VMEM_SHARED((NP, 16), jnp.float32),
            pltpu.SemaphoreType.DMA,
            pltpu.SemaphoreType.DMA,
        ],
        compiler_params=pltpu.CompilerParams(
            needs_layout_passes=False, use_tc_tiling_on_sc=False),
    )
    def ek(xl_hbm, xr_hbm, src_hbm, dst_hbm, att_hbm, zmsg_hbm, zden_hbm,
           outm_hbm, outd_hbm,
           src_v, dst_v, xl_rows, xr_rows, msgbuf, exbuf, att_v,
           msg_acc, den_acc, sem_a, sem_b):
        cid = lax.axis_index("c")
        sid = lax.axis_index("s")
        wid = sid * 2 + cid
        iota16 = lax.broadcasted_iota(jnp.int32, (16,), 0)
        r0 = sid * RPT

        pltpu.sync_copy(zmsg_hbm.at[pl.ds(r0, RPT)], msg_acc.at[pl.ds(r0, RPT)])
        pltpu.sync_copy(zden_hbm.at[pl.ds(r0, RPT)], den_acc.at[pl.ds(r0, RPT)])
        pltpu.sync_copy(att_hbm, att_v)

        def zb(i, c):
            plsc.store_scatter(exbuf, [jnp.full((16,), i, jnp.int32), iota16],
                               jnp.zeros((16,), jnp.float32))
            return c
        lax.fori_loop(0, CH, zb, 0)
        plsc.subcore_barrier()

        def chunk(ci, carry):
            off = wid * EPT + ci * CH
            pltpu.sync_copy(src_hbm.at[pl.ds(off, CH)], src_v)
            pltpu.sync_copy(dst_hbm.at[pl.ds(off, CH)], dst_v)
            cpa = pltpu.async_copy(xl_hbm.at[src_v], xl_rows, sem_a)
            cpb = pltpu.async_copy(xr_hbm.at[dst_v], xr_rows, sem_b)
            cpa.wait()
            cpb.wait()

            def group(g, c2):
                row = g * 16 + iota16
                for h in range(H):
                    def cbody(c, acc):
                        colk = jnp.full((16,), h * CPH + c, jnp.int32)
                        a = plsc.load_gather(xl_rows, [row, colk])
                        b = plsc.load_gather(xr_rows, [row, colk])
                        attk = plsc.load_gather(att_v, [colk])
                        m = a + b
                        lr = jnp.maximum(m, 0.2 * m)
                        return acc + lr * attk
                    acc = lax.fori_loop(0, CPH, cbody,
                                        jnp.zeros((16,), jnp.float32))
                    exh = jnp.exp(acc)
                    plsc.store_scatter(
                        exbuf, [row, jnp.full((16,), h, jnp.int32)], exh)

                    def mbody(c, c3):
                        colk = jnp.full((16,), h * CPH + c, jnp.int32)
                        a = plsc.load_gather(xl_rows, [row, colk])
                        plsc.store_scatter(msgbuf, [row, colk], a * exh)
                        return c3
                    lax.fori_loop(0, CPH, mbody, 0)
                return c2
            lax.fori_loop(0, CH // 16, group, 0)

            pltpu.sync_copy(msgbuf, msg_acc.at[dst_v], add=True)
            pltpu.sync_copy(exbuf, den_acc.at[dst_v], add=True)
            return carry
        lax.fori_loop(0, n_chunks, chunk, 0)

        plsc.subcore_barrier()
        pltpu.sync_copy(msg_acc.at[pl.ds(r0, RPT)],
                        outm_hbm.at[cid].at[pl.ds(r0, RPT)])
        pltpu.sync_copy(den_acc.at[pl.ds(r0, RPT)],
                        outd_hbm.at[cid].at[pl.ds(r0, RPT)])

    return ek(xl, xr, src, dst, att, zmsg, zden)


def _mid_kernel(N, mp_ref, dp_ref, e_ref, b_ref, wl_ref, wr_ref,
                ol_ref, or_ref):
    s = mp_ref[0] + mp_ref[1]
    d = dp_ref[0] + dp_ref[1]
    recip = 1.0 / (d + 1e-16)
    rexp = jnp.dot(recip, e_ref[...], preferred_element_type=jnp.float32)
    h = s * rexp + b_ref[...]
    h = jnp.where(h > 0, h, jnp.exp(h) - 1.0)
    rows = lax.broadcasted_iota(jnp.int32, h.shape, 0)
    h = jnp.where(rows < N, h, 0.0)
    ol_ref[...] = jnp.dot(h, wl_ref[...], preferred_element_type=jnp.float32)
    or_ref[...] = jnp.dot(h, wr_ref[...], preferred_element_type=jnp.float32)


def _mid(N, NP, mp, dp, expand, b, wl, wr):
    return pl.pallas_call(
        functools.partial(_mid_kernel, N),
        out_shape=[jax.ShapeDtypeStruct((NP, _F), jnp.float32)] * 2,
    )(mp, dp, expand, b, wl, wr)


def _post_kernel(mp_ref, dp_ref, e_ref, b_ref, o_ref):
    s = mp_ref[0] + mp_ref[1]
    d = dp_ref[0] + dp_ref[1]
    recip = 1.0 / (d + 1e-16)
    rexp = jnp.dot(recip, e_ref[...], preferred_element_type=jnp.float32)
    h = s * rexp + b_ref[...]
    h = jnp.where(h > 0, h, jnp.exp(h) - 1.0)
    m = jnp.max(h, axis=-1, keepdims=True)
    z = h - m
    lse = jnp.log(jnp.sum(jnp.exp(z), axis=-1, keepdims=True))
    o_ref[...] = z - lse


def _post(NP, mp, dp, expand, b):
    return pl.pallas_call(
        _post_kernel,
        out_shape=jax.ShapeDtypeStruct((NP, _F), jnp.float32),
    )(mp, dp, expand, b)


def kernel(x, edge_index, W1_l, W1_r, att1, b1, W2_l, W2_r, att2, b2):
    N = x.shape[0]
    NP = ((N + 1 + 127) // 128) * 128
    E2 = edge_index.shape[1] + N
    n_chunks = -(-E2 // (_NW * _CH))
    E_pad = n_chunks * _NW * _CH

    loop = jnp.arange(N, dtype=edge_index.dtype)
    epad = jnp.full((E_pad - E2,), N, edge_index.dtype)
    src = jnp.concatenate([edge_index[0], loop, epad])
    dst = jnp.concatenate([edge_index[1], loop, epad])

    zmsg = jnp.zeros((NP, _F), jnp.float32)
    zden = jnp.zeros((NP, 16), jnp.float32)
    cols = jnp.arange(_F)[None, :]
    rows16 = jnp.arange(16)[:, None]
    exp_h8 = ((cols // 8 == rows16) & (rows16 < 8)).astype(jnp.float32)
    exp_h1 = (rows16 == 0).astype(jnp.float32) * jnp.ones((1, _F), jnp.float32)

    xl1, xr1 = _mm2(x, W1_l, W1_r)
    padn = ((0, NP - N), (0, 0))
    xl1 = jnp.pad(xl1, padn)
    xr1 = jnp.pad(xr1, padn)

    mp1, dp1 = _sc_edge_pass(8, NP, n_chunks, xl1, xr1, src, dst,
                             att1.reshape(-1), zmsg, zden)
    xl2, xr2 = _mid(N, NP, mp1, dp1, exp_h8, b1.reshape(1, -1), W2_l, W2_r)
    mp2, dp2 = _sc_edge_pass(1, NP, n_chunks, xl2, xr2, src, dst,
                             att2.reshape(-1), zmsg, zden)
    out = _post(NP, mp2, dp2, exp_h1, b2.reshape(1, -1))
    return out[:N]


# speedup vs baseline: 18.7988x; 1.3430x over previous
import functools

import jax
import jax.numpy as jnp
from jax import lax
from jax.experimental import pallas as pl
from jax.experimental.pallas import tpu as pltpu
from jax.experimental.pallas import tpu_sc as plsc

_F = 64
_CH = 128
_NW = 32


def _mm2_kernel(x_ref, wl_ref, wr_ref, ol_ref, or_ref):
    x = x_ref[...]
    ol_ref[...] = jnp.dot(x, wl_ref[...], preferred_element_type=jnp.float32,
                          precision=jax.lax.Precision.HIGHEST)
    or_ref[...] = jnp.dot(x, wr_ref[...], preferred_element_type=jnp.float32,
                          precision=jax.lax.Precision.HIGHEST)


def _mm2(x, wl, wr):
    n = x.shape[0]
    f = wl.shape[1]
    return pl.pallas_call(
        _mm2_kernel,
        out_shape=[jax.ShapeDtypeStruct((n, f), jnp.float32)] * 2,
    )(x, wl, wr)


def _sc_edge_pass(H, NP, n_chunks, xl, xr, srcm, dstm, att, zmsg, zden):
    CH = _CH
    RPT = NP // 16
    CPH = _F // H
    mesh = plsc.VectorSubcoreMesh(core_axis_name="c", subcore_axis_name="s")

    @functools.partial(
        pl.kernel,
        mesh=mesh,
        out_type=[
            jax.ShapeDtypeStruct((2, NP, _F), jnp.float32),
            jax.ShapeDtypeStruct((2, NP, 16), jnp.float32),
        ],
        scratch_types=[
            pltpu.VMEM((_F,), jnp.float32),
            pltpu.VMEM((CH, _F), jnp.float32),
            pltpu.VMEM((CH, _F), jnp.float32),
            pltpu.VMEM((CH, _F), jnp.float32),
            pltpu.VMEM((CH, _F), jnp.float32),
            pltpu.VMEM((CH, _F), jnp.float32),
            pltpu.VMEM((CH, _F), jnp.float32),
            pltpu.VMEM((CH, 16), jnp.float32),
            pltpu.VMEM((CH, 16), jnp.float32),
            pltpu.VMEM((CH,), jnp.int32),
            pltpu.VMEM((CH,), jnp.int32),
            pltpu.VMEM((CH,), jnp.int32),
            pltpu.VMEM((CH,), jnp.int32),
            pltpu.VMEM((CH,), jnp.int32),
            pltpu.VMEM((CH,), jnp.int32),
            pltpu.VMEM((CH,), jnp.int32),
            pltpu.VMEM((CH,), jnp.int32),
            pltpu.VMEM_SHARED((NP, _F), jnp.float32),
            pltpu.VMEM_SHARED((NP, 16), jnp.float32),
            pltpu.SemaphoreType.DMA,
            pltpu.SemaphoreType.DMA,
            pltpu.SemaphoreType.DMA,
            pltpu.SemaphoreType.DMA,
            pltpu.SemaphoreType.DMA,
            pltpu.SemaphoreType.DMA,
            pltpu.SemaphoreType.DMA,
            pltpu.SemaphoreType.DMA,
            pltpu.SemaphoreType.DMA,
            pltpu.SemaphoreType.DMA,
        ],
        compiler_params=pltpu.CompilerParams(
            needs_layout_passes=False, use_tc_tiling_on_sc=False),
    )
    def ek(xl_hbm, xr_hbm, srcm_hbm, dstm_hbm, att_hbm, zmsg_hbm, zden_hbm,
           outm_hbm, outd_hbm,
           att_v, xl0, xl1, xr0, xr1, mg0, mg1, ex0, ex1,
           si0, si1, si2, si3, di0, di1, di2, di3,
           macc, dacc, sga0, sga1, sgb0, sgb1,
           ssm0, ssm1, sse0, sse1, ssi, sdi):
        cid = lax.axis_index("c")
        sid = lax.axis_index("s")
        wid = sid * 2 + cid
        iota16 = lax.broadcasted_iota(jnp.int32, (16,), 0)
        r0 = sid * RPT

        pltpu.sync_copy(zmsg_hbm.at[pl.ds(r0, RPT)], macc.at[pl.ds(r0, RPT)])
        pltpu.sync_copy(zden_hbm.at[pl.ds(r0, RPT)], dacc.at[pl.ds(r0, RPT)])
        pltpu.sync_copy(att_hbm, att_v)

        def zb(i, c):
            for exb in (ex0, ex1):
                plsc.store_scatter(
                    exb, [jnp.full((16,), i, jnp.int32), iota16],
                    jnp.zeros((16,), jnp.float32))
            return c
        lax.fori_loop(0, CH, zb, 0)

        sis = (si0, si1, si2, si3)
        dis = (di0, di1, di2, di3)
        pltpu.sync_copy(srcm_hbm.at[wid].at[0], si0)
        pltpu.sync_copy(dstm_hbm.at[wid].at[0], di0)
        pltpu.sync_copy(srcm_hbm.at[wid].at[1], si1)
        pltpu.sync_copy(dstm_hbm.at[wid].at[1], di1)
        pltpu.async_copy(xl_hbm.at[si0], xl0, sga0)
        pltpu.async_copy(xr_hbm.at[di0], xr0, sgb0)
        pltpu.async_copy(xl_hbm.at[si1], xl1, sga1)
        pltpu.async_copy(xr_hbm.at[di1], xr1, sgb1)

        plsc.subcore_barrier()

        def compute(xlb, xrb, mgb, exb):
            if H >= 4:
                for h in range(H):
                    atts = [plsc.load_gather(
                        att_v, [jnp.full((16,), h * CPH + c, jnp.int32)])
                        for c in range(CPH)]

                    def ghead(g, c2, h=h, atts=atts):
                        row = g * 16 + iota16
                        accv = jnp.zeros((16,), jnp.float32)
                        avals = []
                        for c in range(CPH):
                            colk = jnp.full((16,), h * CPH + c, jnp.int32)
                            a = plsc.load_gather(xlb, [row, colk])
                            b = plsc.load_gather(xrb, [row, colk])
                            m = a + b
                            accv = accv + jnp.maximum(m, 0.2 * m) * atts[c]
                            avals.append(a)
                        exh = jnp.exp(accv)
                        plsc.store_scatter(
                            exb, [row, jnp.full((16,), h, jnp.int32)], exh)
                        for c in range(CPH):
                            colk = jnp.full((16,), h * CPH + c, jnp.int32)
                            plsc.store_scatter(mgb, [row, colk],
                                               avals[c] * exh)
                        return c2
                    lax.fori_loop(0, CH // 16, ghead, 0)
            else:
                def group(g, c2):
                    row = g * 16 + iota16
                    for h in range(H):
                        accv = jnp.zeros((16,), jnp.float32)
                        for c in range(CPH):
                            colk = jnp.full((16,), h * CPH + c, jnp.int32)
                            a = plsc.load_gather(xlb, [row, colk])
                            b = plsc.load_gather(xrb, [row, colk])
                            attk = plsc.load_gather(att_v, [colk])
                            m = a + b
                            accv = accv + jnp.maximum(m, 0.2 * m) * attk
                        exh = jnp.exp(accv)
                        plsc.store_scatter(
                            exb, [row, jnp.full((16,), h, jnp.int32)], exh)
                        for c in range(CPH):
                            colk = jnp.full((16,), h * CPH + c, jnp.int32)
                            a = plsc.load_gather(xlb, [row, colk])
                            plsc.store_scatter(mgb, [row, colk], a * exh)
                    return c2
                lax.fori_loop(0, CH // 16, group, 0)

        dslots = ((xl0, xr0, mg0, ex0, sga0, sgb0, ssm0, sse0),
                  (xl1, xr1, mg1, ex1, sga1, sgb1, ssm1, sse1))

        def quad(ci4, carry):
            for j in range(4):
                xlb, xrb, mgb, exb, sga, sgb, ssm, sse = dslots[j % 2]
                ci = ci4 * 4 + j
                pltpu.make_async_copy(xl_hbm.at[sis[j]], xlb, sga).wait()
                pltpu.make_async_copy(xr_hbm.at[dis[j]], xrb, sgb).wait()

                @pl.when(ci >= 2)
                def _():
                    pltpu.make_async_copy(
                        mgb, macc.at[dis[(j + 2) % 4]], ssm).wait()
                    pltpu.make_async_copy(
                        exb, dacc.at[dis[(j + 2) % 4]], sse).wait()

                @pl.when(ci + 2 < n_chunks)
                def _():
                    pltpu.async_copy(
                        srcm_hbm.at[wid].at[ci + 2], sis[(j + 2) % 4], ssi)
                    pltpu.async_copy(
                        dstm_hbm.at[wid].at[ci + 2], dis[(j + 2) % 4], sdi)

                compute(xlb, xrb, mgb, exb)
                pltpu.async_copy(mgb, macc.at[dis[j]], ssm, add=True)
                pltpu.async_copy(exb, dacc.at[dis[j]], sse, add=True)

                @pl.when(ci + 2 < n_chunks)
                def _():
                    pltpu.make_async_copy(
                        srcm_hbm.at[wid].at[ci + 2], sis[(j + 2) % 4],
                        ssi).wait()
                    pltpu.make_async_copy(
                        dstm_hbm.at[wid].at[ci + 2], dis[(j + 2) % 4],
                        sdi).wait()
                    pltpu.async_copy(xl_hbm.at[sis[(j + 2) % 4]], xlb, sga)
                    pltpu.async_copy(xr_hbm.at[dis[(j + 2) % 4]], xrb, sgb)
            return carry
        lax.fori_loop(0, n_chunks // 4, quad, 0)

        pltpu.make_async_copy(mg0, macc.at[di2], ssm0).wait()
        pltpu.make_async_copy(ex0, dacc.at[di2], sse0).wait()
        pltpu.make_async_copy(mg1, macc.at[di3], ssm1).wait()
        pltpu.make_async_copy(ex1, dacc.at[di3], sse1).wait()
        plsc.subcore_barrier()
        pltpu.sync_copy(macc.at[pl.ds(r0, RPT)],
                        outm_hbm.at[cid].at[pl.ds(r0, RPT)])
        pltpu.sync_copy(dacc.at[pl.ds(r0, RPT)],
                        outd_hbm.at[cid].at[pl.ds(r0, RPT)])

    return ek(xl, xr, srcm, dstm, att, zmsg, zden)


def _mid_kernel(N, mp_ref, dp_ref, e_ref, b_ref, wl_ref, wr_ref,
                ol_ref, or_ref):
    s = mp_ref[0] + mp_ref[1]
    d = dp_ref[0] + dp_ref[1]
    recip = 1.0 / (d + 1e-16)
    rexp = jnp.dot(recip, e_ref[...], preferred_element_type=jnp.float32)
    h = s * rexp + b_ref[...]
    h = jnp.where(h > 0, h, jnp.exp(h) - 1.0)
    rows = lax.broadcasted_iota(jnp.int32, h.shape, 0)
    h = jnp.where(rows < N, h, 0.0)
    ol_ref[...] = jnp.dot(h, wl_ref[...], preferred_element_type=jnp.float32,
                          precision=jax.lax.Precision.HIGHEST)
    or_ref[...] = jnp.dot(h, wr_ref[...], preferred_element_type=jnp.float32,
                          precision=jax.lax.Precision.HIGHEST)


def _mid(N, NP, mp, dp, expand, b, wl, wr):
    return pl.pallas_call(
        functools.partial(_mid_kernel, N),
        out_shape=[jax.ShapeDtypeStruct((NP, _F), jnp.float32)] * 2,
    )(mp, dp, expand, b, wl, wr)


def _post_kernel(mp_ref, dp_ref, e_ref, b_ref, o_ref):
    s = mp_ref[0] + mp_ref[1]
    d = dp_ref[0] + dp_ref[1]
    recip = 1.0 / (d + 1e-16)
    rexp = jnp.dot(recip, e_ref[...], preferred_element_type=jnp.float32)
    h = s * rexp + b_ref[...]
    h = jnp.where(h > 0, h, jnp.exp(h) - 1.0)
    m = jnp.max(h, axis=-1, keepdims=True)
    z = h - m
    lse = jnp.log(jnp.sum(jnp.exp(z), axis=-1, keepdims=True))
    o_ref[...] = z - lse


def _post(NP, mp, dp, expand, b):
    return pl.pallas_call(
        _post_kernel,
        out_shape=jax.ShapeDtypeStruct((NP, _F), jnp.float32),
    )(mp, dp, expand, b)


def kernel(x, edge_index, W1_l, W1_r, att1, b1, W2_l, W2_r, att2, b2):
    N = x.shape[0]
    NP = ((N + 1 + 127) // 128) * 128
    E2 = edge_index.shape[1] + N
    n_chunks = -(-E2 // (_NW * _CH))
    n_chunks = ((n_chunks + 3) // 4) * 4
    E_pad = n_chunks * _NW * _CH

    loop = jnp.arange(N, dtype=edge_index.dtype)
    epad = jnp.full((E_pad - E2,), N, edge_index.dtype)
    src = jnp.concatenate([edge_index[0], loop, epad])
    dst = jnp.concatenate([edge_index[1], loop, epad])
    srcm = src.reshape(_NW, n_chunks, _CH)
    dstm = dst.reshape(_NW, n_chunks, _CH)

    zmsg = jnp.zeros((NP, _F), jnp.float32)
    zden = jnp.zeros((NP, 16), jnp.float32)
    cols = jnp.arange(_F)[None, :]
    rows16 = jnp.arange(16)[:, None]
    exp_h8 = ((cols // 8 == rows16) & (rows16 < 8)).astype(jnp.float32)
    exp_h1 = (rows16 == 0).astype(jnp.float32) * jnp.ones((1, _F), jnp.float32)

    xl1, xr1 = _mm2(x, W1_l, W1_r)
    padn = ((0, NP - N), (0, 0))
    xl1 = jnp.pad(xl1, padn)
    xr1 = jnp.pad(xr1, padn)

    mp1, dp1 = _sc_edge_pass(8, NP, n_chunks, xl1, xr1, srcm, dstm,
                             att1.reshape(-1), zmsg, zden)
    xl2, xr2 = _mid(N, NP, mp1, dp1, exp_h8, b1.reshape(1, -1), W2_l, W2_r)
    mp2, dp2 = _sc_edge_pass(1, NP, n_chunks, xl2, xr2, srcm, dstm,
                             att2.reshape(-1), zmsg, zden)
    out = _post(NP, mp2, dp2, exp_h1, b2.reshape(1, -1))
    return out[:N]
